# baseline (speedup 1.0000x reference)
import functools

import jax
import jax.numpy as jnp
import numpy as np
from jax import lax
from jax.experimental import pallas as pl
from jax.experimental.pallas import tpu as pltpu
from jax.experimental.pallas import tpu_sc as plsc

B, F, D = 16384, 26, 16
N = B * F
_info = plsc.get_sparse_core_info()
NC, NS, L = _info.num_cores, _info.num_subcores, _info.num_lanes
NW = NC * NS
NPW = N // NW
PERIOD = (F * L) // np.gcd(F, L)
NSEG = PERIOD // L
NITER = NPW // PERIOD
CHUNK = 3328
NCHUNK = NPW // CHUNK

_FIELD_DIMS = [100000] * F
_OFFSETS = np.concatenate([[0], np.cumsum(_FIELD_DIMS)[:-1]]).astype(np.int32)
_PATTERN = _OFFSETS[np.arange(PERIOD) % F]


def _sc_kernel(x_hbm, patt_hbm, table_hbm, out_hbm, idx_v, patt_v, rows_v, sem):
    wid = lax.axis_index("s") * NC + lax.axis_index("c")
    base = wid * NPW

    pltpu.sync_copy(x_hbm.at[pl.ds(base, NPW)], idx_v)
    pltpu.sync_copy(patt_hbm, patt_v)

    pregs = [patt_v[pl.ds(u * L, L)] for u in range(NSEG)]

    def add_offsets(t, carry):
        s = t * PERIOD
        for u in range(NSEG):
            sl = pl.ds(s + u * L, L)
            idx_v[sl] = idx_v[sl] + pregs[u]
        return carry

    lax.fori_loop(0, NITER, add_offsets, 0)

    def do_chunk(k, carry):
        pltpu.async_copy(
            table_hbm.at[idx_v.at[pl.ds(k * CHUNK, CHUNK)]], rows_v, sem
        ).wait()
        pltpu.sync_copy(rows_v, out_hbm.at[pl.ds(base + k * CHUNK, CHUNK)])
        return carry

    lax.fori_loop(0, NCHUNK, do_chunk, 0)


@jax.jit
def _run(x_flat, patt, table):
    return pl.kernel(
        _sc_kernel,
        out_type=jax.ShapeDtypeStruct((N, D), jnp.float32),
        mesh=plsc.VectorSubcoreMesh(core_axis_name="c", subcore_axis_name="s"),
        scratch_types=[
            pltpu.VMEM((NPW,), jnp.int32),
            pltpu.VMEM((PERIOD,), jnp.int32),
            pltpu.VMEM((CHUNK, D), jnp.float32),
            pltpu.SemaphoreType.DMA,
        ],
        compiler_params=pltpu.CompilerParams(use_tc_tiling_on_sc=False),
    )(x_flat, patt, table)


def kernel(x, table):
    patt = jnp.asarray(_PATTERN)
    out = _run(x.reshape(-1), patt, table)
    return out.reshape(B, F, D)
